# baseline (speedup 1.0000x reference)
import jax
import jax.numpy as jnp
from jax import lax
from jax.experimental import pallas as pl
from jax.experimental.pallas import tpu as pltpu
from jax.experimental.pallas import tpu_sc as plsc

B = 4
S = 2048
DIM = 1024
N = B * S
NC = 2
NS = 16
NW = NC * NS
SPW = S // NW
SCH = 8
NCHUNKS = SPW // SCH
GCH = B * SCH
ROWS_PER_W = B * SPW
LANES = 16


def _body(x_ref, tok_ref, pos_ref, out_ref,
          idx_v, tok0, tok1, pos0, pos1, sem_g, sem_p, sem_o):
    wid = lax.axis_index("s") * NC + lax.axis_index("c")
    sbase = wid * SPW
    tok_bufs = (tok0, tok1)
    pos_bufs = (pos0, pos1)

    ld = [pltpu.async_copy(x_ref.at[pl.ds(b * S + sbase + c * SCH, SCH)],
                           idx_v.at[pl.ds(c * GCH + b * SCH, SCH)], sem_p)
          for c in range(NCHUNKS) for b in range(B)]
    for cp in ld:
        cp.wait()

    def start_chunk(c, tbuf, pbuf):
        pltpu.async_copy(tok_ref.at[idx_v.at[pl.ds(c * GCH, GCH)]], tbuf, sem_g)
        pltpu.async_copy(pos_ref.at[pl.ds(sbase + c * SCH, SCH)], pbuf, sem_p)

    def wait_in(tbuf, pbuf):
        pltpu.make_async_copy(
            tok_ref.at[idx_v.at[pl.ds(0, GCH)]], tbuf, sem_g).wait()
        pltpu.make_async_copy(pos_ref.at[pl.ds(0, SCH)], pbuf, sem_p).wait()

    def drain_out(tbuf):
        for b in range(B):
            pltpu.make_async_copy(
                tbuf.at[pl.ds(b * SCH, SCH)],
                out_ref.at[pl.ds(0, SCH)], sem_o).wait()

    def compute(tbuf, pbuf):
        @pl.loop(0, SCH)
        def _row(r):
            for j in range(DIM // LANES):
                sl = pl.ds(j * LANES, LANES)
                pv = pbuf[r, sl]
                for b in range(B):
                    tbuf[b * SCH + r, sl] = tbuf[b * SCH + r, sl] + pv

    def write_out(c, tbuf):
        for b in range(B):
            pltpu.async_copy(
                tbuf.at[pl.ds(b * SCH, SCH)],
                out_ref.at[pl.ds(b * S + sbase + c * SCH, SCH)], sem_o)

    start_chunk(0, tok0, pos0)

    @pl.loop(0, NCHUNKS // 2)
    def _cc(cc):
        for par in (0, 1):
            c = cc * 2 + par
            tbuf, pbuf = tok_bufs[par], pos_bufs[par]
            ntbuf, npbuf = tok_bufs[1 - par], pos_bufs[1 - par]

            @pl.when(c >= 1)
            def _():
                drain_out(ntbuf)

            @pl.when(c + 1 < NCHUNKS)
            def _():
                start_chunk(c + 1, ntbuf, npbuf)

            wait_in(tbuf, pbuf)
            compute(tbuf, pbuf)
            write_out(c, tbuf)

    drain_out(tok_bufs[(NCHUNKS - 1) % 2])


@jax.jit
def _run(xg, token_table, pos_table):
    mesh = plsc.VectorSubcoreMesh(core_axis_name="c", subcore_axis_name="s")
    return pl.kernel(
        _body,
        out_type=jax.ShapeDtypeStruct((N, DIM), jnp.float32),
        mesh=mesh,
        scratch_types=[
            pltpu.VMEM((ROWS_PER_W,), jnp.int32),
            pltpu.VMEM((GCH, DIM), jnp.float32),
            pltpu.VMEM((GCH, DIM), jnp.float32),
            pltpu.VMEM((SCH, DIM), jnp.float32),
            pltpu.VMEM((SCH, DIM), jnp.float32),
            pltpu.SemaphoreType.DMA,
            pltpu.SemaphoreType.DMA,
            pltpu.SemaphoreType.DMA,
        ],
    )(xg, token_table, pos_table)


def kernel(x, token_table, pos_table):
    xg = x.astype(jnp.int32).reshape(N)
    out = _run(xg, token_table, pos_table)
    return out.reshape(B, S, DIM)


# speedup vs baseline: 1.0113x; 1.0113x over previous
import jax
import jax.numpy as jnp
from jax import lax
from jax.experimental import pallas as pl
from jax.experimental.pallas import tpu as pltpu
from jax.experimental.pallas import tpu_sc as plsc

B = 4
S = 2048
DIM = 1024
N = B * S
NC = 2
NS = 16
NW = NC * NS
SPW = S // NW
SCH = 8
NCHUNKS = SPW // SCH
GCH = B * SCH
ROWS_PER_W = B * SPW
LANES = 16


def _body(xg_ref, tok_ref, pos_ref, out_ref,
          idx_v, tok0, tok1, pos0, pos1, sem_g, sem_p, sem_o):
    wid = lax.axis_index("s") * NC + lax.axis_index("c")
    ibase = wid * ROWS_PER_W
    sbase = wid * SPW
    tok_bufs = (tok0, tok1)
    pos_bufs = (pos0, pos1)

    pltpu.sync_copy(xg_ref.at[pl.ds(ibase, ROWS_PER_W)], idx_v)

    def start_chunk(c, tbuf, pbuf):
        pltpu.async_copy(tok_ref.at[idx_v.at[pl.ds(c * GCH, GCH)]], tbuf, sem_g)
        pltpu.async_copy(pos_ref.at[pl.ds(sbase + c * SCH, SCH)], pbuf, sem_p)

    def wait_in(tbuf, pbuf):
        pltpu.make_async_copy(
            tok_ref.at[idx_v.at[pl.ds(0, GCH)]], tbuf, sem_g).wait()
        pltpu.make_async_copy(pos_ref.at[pl.ds(0, SCH)], pbuf, sem_p).wait()

    def drain_out(tbuf):
        for b in range(B):
            pltpu.make_async_copy(
                tbuf.at[pl.ds(b * SCH, SCH)],
                out_ref.at[pl.ds(0, SCH)], sem_o).wait()

    def compute(tbuf, pbuf):
        @pl.loop(0, SCH)
        def _row(r):
            for j in range(DIM // LANES):
                sl = pl.ds(j * LANES, LANES)
                pv = pbuf[r, sl]
                for b in range(B):
                    tbuf[b * SCH + r, sl] = tbuf[b * SCH + r, sl] + pv

    def write_out(c, tbuf):
        for b in range(B):
            pltpu.async_copy(
                tbuf.at[pl.ds(b * SCH, SCH)],
                out_ref.at[pl.ds(b * S + sbase + c * SCH, SCH)], sem_o)

    start_chunk(0, tok0, pos0)

    @pl.loop(0, NCHUNKS // 2)
    def _cc(cc):
        for par in (0, 1):
            c = cc * 2 + par
            tbuf, pbuf = tok_bufs[par], pos_bufs[par]
            ntbuf, npbuf = tok_bufs[1 - par], pos_bufs[1 - par]

            @pl.when(c >= 1)
            def _():
                drain_out(ntbuf)

            @pl.when(c + 1 < NCHUNKS)
            def _():
                start_chunk(c + 1, ntbuf, npbuf)

            wait_in(tbuf, pbuf)
            compute(tbuf, pbuf)
            write_out(c, tbuf)

    drain_out(tok_bufs[(NCHUNKS - 1) % 2])


@jax.jit
def _run(xg, token_table, pos_table):
    mesh = plsc.VectorSubcoreMesh(core_axis_name="c", subcore_axis_name="s")
    return pl.kernel(
        _body,
        out_type=jax.ShapeDtypeStruct((N, DIM), jnp.float32),
        mesh=mesh,
        scratch_types=[
            pltpu.VMEM((ROWS_PER_W,), jnp.int32),
            pltpu.VMEM((GCH, DIM), jnp.float32),
            pltpu.VMEM((GCH, DIM), jnp.float32),
            pltpu.VMEM((SCH, DIM), jnp.float32),
            pltpu.VMEM((SCH, DIM), jnp.float32),
            pltpu.SemaphoreType.DMA,
            pltpu.SemaphoreType.DMA,
            pltpu.SemaphoreType.DMA,
        ],
    )(xg, token_table, pos_table)


def kernel(x, token_table, pos_table):
    xg = (x.astype(jnp.int32)
           .reshape(B, NW, NCHUNKS, SCH)
           .transpose(1, 2, 0, 3)
           .reshape(N))
    out = _run(xg, token_table, pos_table)
    return out.reshape(B, S, DIM)
